# baseline (speedup 1.0000x reference)
import functools

import jax
import jax.numpy as jnp
from jax import lax
from jax.experimental import pallas as pl
from jax.experimental.pallas import tpu as pltpu
from jax.experimental.pallas import tpu_sc as plsc

NC = 2
NS = 16
L = 16
NW = NC * NS

LANE = 128
VP = 1024
SUB = VP // LANE


def _mtab_body(emb_ref, w_ref, b_ref, m8_ref, lse_ref):
    m = jax.lax.dot_general(
        emb_ref[...], w_ref[...],
        (((1,), (1,)), ((), ())),
        preferred_element_type=jnp.float32,
    ) + b_ref[...]
    V = m.shape[1]
    for j in range(SUB):
        w = min(LANE, V - j * LANE)
        m8_ref[:, j, :w] = m[:, j * LANE:j * LANE + w]
    mx = jnp.max(m, axis=1, keepdims=True)
    lse_ref[...] = mx + jnp.log(jnp.sum(jnp.exp(m - mx), axis=1, keepdims=True))


def _make_mtab(V):
    return pl.pallas_call(
        _mtab_body,
        out_shape=(
            jax.ShapeDtypeStruct((V, SUB, LANE), jnp.float32),
            jax.ShapeDtypeStruct((V, 1), jnp.float32),
        ),
    )


def _make_gather(V, NTOK):
    TPT = NTOK // NW
    CH = 64
    assert TPT % CH == 0 and CH % L == 0
    mesh = plsc.VectorSubcoreMesh(core_axis_name="c", subcore_axis_name="s")

    @functools.partial(
        pl.kernel,
        mesh=mesh,
        compiler_params=pltpu.CompilerParams(
            use_tc_tiling_on_sc=False, needs_layout_passes=False),
        out_type=(
            jax.ShapeDtypeStruct((NTOK, SUB, LANE), jnp.float32),
            jax.ShapeDtypeStruct((NW * L,), jnp.float32),
        ),
        scratch_types=[
            pltpu.VMEM((TPT,), jnp.int32),
            pltpu.VMEM((TPT,), jnp.int32),
            pltpu.VMEM((V,), jnp.float32),
            pltpu.VMEM((CH, SUB, LANE), jnp.float32),
            pltpu.VMEM((L,), jnp.float32),
            pltpu.SemaphoreType.DMA,
        ],
    )
    def gather_k(m8_hbm, idx_hbm, tgt_hbm, lse_hbm, out_hbm, part_hbm,
                 idx_v, tgt_v, lse_v, rows_v, acc_v, sem):
        wid = lax.axis_index("s") * NC + lax.axis_index("c")
        base = wid * TPT
        pltpu.sync_copy(idx_hbm.at[pl.ds(base, TPT)], idx_v)
        pltpu.sync_copy(tgt_hbm.at[pl.ds(base, TPT)], tgt_v)
        pltpu.sync_copy(lse_hbm, lse_v)
        acc = jnp.zeros((L,), jnp.float32)
        for c in range(TPT // CH):
            pltpu.async_copy(
                m8_hbm.at[idx_v.at[pl.ds(c * CH, CH)]], rows_v, sem).wait()
            pltpu.sync_copy(rows_v, out_hbm.at[pl.ds(base + c * CH, CH)])
            for g in range(CH // L):
                off = c * CH + g * L
                toks = idx_v[pl.ds(off, L)]
                tgts = tgt_v[pl.ds(off, L)]
                rid = lax.iota(jnp.int32, L) + (g * L)
                tlogit = plsc.load_gather(
                    rows_v, [rid, tgts >> 7, tgts & (LANE - 1)])
                ltok = plsc.load_gather(lse_v, [toks])
                acc = acc + (ltok - tlogit)
        acc_v[...] = acc
        pltpu.sync_copy(acc_v, part_hbm.at[pl.ds(wid * L, L)])

    return gather_k


def _fin_body(in_ref, out_ref, *, V):
    for j in range(SUB):
        w = min(LANE, V - j * LANE)
        out_ref[0, :, pl.ds(j * LANE, w)] = in_ref[:, j, :w]


def _make_finisher(V, Bv, Sv):
    RB = 256
    grid = (Bv, Sv // RB)
    return pl.pallas_call(
        functools.partial(_fin_body, V=V),
        grid=grid,
        in_specs=[pl.BlockSpec(
            (RB, SUB, LANE), lambda b, s: (b * (Sv // RB) + s, 0, 0))],
        out_specs=pl.BlockSpec((1, RB, V), lambda b, s: (b, s, 0)),
        out_shape=jax.ShapeDtypeStruct((Bv, Sv, V), jnp.float32),
    )


def _loss_body(p_ref, o_ref, *, ntok):
    o_ref[...] = jnp.sum(p_ref[...], keepdims=True) * (1.0 / ntok)


def _make_loss(ntok):
    return pl.pallas_call(
        functools.partial(_loss_body, ntok=ntok),
        out_shape=jax.ShapeDtypeStruct((1, 1), jnp.float32),
    )


def kernel(input, target, emb_table, W, b):
    Bv, Sv = input.shape
    V, E = emb_table.shape
    NTOK = Bv * Sv

    m8, lse = _make_mtab(V)(emb_table, W, b.reshape(1, V))
    idx = input.reshape(NTOK)
    tgt = target.reshape(NTOK)
    out8, part = _make_gather(V, NTOK)(m8, idx, tgt, lse.reshape(V))
    logits = _make_finisher(V, Bv, Sv)(out8)
    loss2d = _make_loss(NTOK)(part.reshape(NW, L))
    return logits, loss2d[0, 0]


# speedup vs baseline: 1.1313x; 1.1313x over previous
import functools

import jax
import jax.numpy as jnp
from jax import lax
from jax.experimental import pallas as pl
from jax.experimental.pallas import tpu as pltpu
from jax.experimental.pallas import tpu_sc as plsc

NC = 2
NS = 16
L = 16
NW = NC * NS

LANE = 128
VP = 1024
SUB = VP // LANE


def _mtab_body(emb_ref, w_ref, b_ref, m8_ref, lse_ref):
    m = jax.lax.dot_general(
        emb_ref[...], w_ref[...],
        (((1,), (1,)), ((), ())),
        preferred_element_type=jnp.float32,
    ) + b_ref[...]
    V = m.shape[1]
    for j in range(SUB):
        w = min(LANE, V - j * LANE)
        m8_ref[:, j, :w] = m[:, j * LANE:j * LANE + w]
    mx = jnp.max(m, axis=1, keepdims=True)
    lse_ref[...] = mx + jnp.log(jnp.sum(jnp.exp(m - mx), axis=1, keepdims=True))


def _make_mtab(V):
    return pl.pallas_call(
        _mtab_body,
        out_shape=(
            jax.ShapeDtypeStruct((V, SUB, LANE), jnp.float32),
            jax.ShapeDtypeStruct((V, 1), jnp.float32),
        ),
    )


def _make_gather(V, NTOK):
    TPT = NTOK // NW
    CH = 64
    assert TPT % CH == 0 and CH % L == 0
    mesh = plsc.VectorSubcoreMesh(core_axis_name="c", subcore_axis_name="s")

    @functools.partial(
        pl.kernel,
        mesh=mesh,
        compiler_params=pltpu.CompilerParams(
            use_tc_tiling_on_sc=False, needs_layout_passes=False),
        out_type=(
            jax.ShapeDtypeStruct((NTOK, SUB, LANE), jnp.float32),
            jax.ShapeDtypeStruct((NW * L,), jnp.float32),
        ),
        scratch_types=[
            pltpu.VMEM((TPT,), jnp.int32),
            pltpu.VMEM((TPT,), jnp.int32),
            pltpu.VMEM((V,), jnp.float32),
            pltpu.VMEM((CH, SUB, LANE), jnp.float32),
            pltpu.VMEM((L,), jnp.float32),
            pltpu.SemaphoreType.DMA,
        ],
    )
    def gather_k(m8_hbm, idx_hbm, tgt_hbm, lse_hbm, out_hbm, part_hbm,
                 idx_v, tgt_v, lse_v, rows_v, acc_v, sem):
        wid = lax.axis_index("s") * NC + lax.axis_index("c")
        base = wid * TPT
        pltpu.sync_copy(idx_hbm.at[pl.ds(base, TPT)], idx_v)
        pltpu.sync_copy(tgt_hbm.at[pl.ds(base, TPT)], tgt_v)
        pltpu.sync_copy(lse_hbm, lse_v)
        acc = jnp.zeros((L,), jnp.float32)
        for c in range(TPT // CH):
            pltpu.async_copy(
                m8_hbm.at[idx_v.at[pl.ds(c * CH, CH)]], rows_v, sem).wait()
            pltpu.sync_copy(rows_v, out_hbm.at[pl.ds(base + c * CH, CH)])
            for g in range(CH // L):
                off = c * CH + g * L
                toks = idx_v[pl.ds(off, L)]
                tgts = tgt_v[pl.ds(off, L)]
                rid = lax.iota(jnp.int32, L) + (g * L)
                tlogit = plsc.load_gather(
                    rows_v, [rid, tgts >> 7, tgts & (LANE - 1)])
                ltok = plsc.load_gather(lse_v, [toks])
                acc = acc + (ltok - tlogit)
        acc_v[...] = acc
        pltpu.sync_copy(acc_v, part_hbm.at[pl.ds(wid * L, L)])

    return gather_k


def _fin_body(in_ref, out_ref, *, V):
    for j in range(SUB):
        w = min(LANE, V - j * LANE)
        out_ref[0, :, pl.ds(j * LANE, w)] = in_ref[:, j, :w]


def _make_finisher(V, Bv, Sv):
    RB = 256
    grid = (Bv, Sv // RB)
    return pl.pallas_call(
        functools.partial(_fin_body, V=V),
        grid=grid,
        in_specs=[pl.BlockSpec(
            (RB, SUB, LANE), lambda b, s: (b * (Sv // RB) + s, 0, 0))],
        out_specs=pl.BlockSpec((1, RB, V), lambda b, s: (b, s, 0)),
        out_shape=jax.ShapeDtypeStruct((Bv, Sv, V), jnp.float32),
    )


def _loss_body(p_ref, o_ref, *, ntok):
    o_ref[...] = jnp.sum(p_ref[...], keepdims=True) * (1.0 / ntok)


def _make_loss(ntok):
    return pl.pallas_call(
        functools.partial(_loss_body, ntok=ntok),
        out_shape=jax.ShapeDtypeStruct((1, 1), jnp.float32),
    )


def kernel(input, target, emb_table, W, b):
    Bv, Sv = input.shape
    V, E = emb_table.shape
    NTOK = Bv * Sv

    m8, lse = _make_mtab(V)(emb_table, W, b.reshape(1, V))
    idx = input.reshape(NTOK)
    tgt = target.reshape(NTOK)
    out8, part = _make_gather(V, NTOK)(m8, idx, tgt, lse.reshape(V))
    logits = out8.reshape(Bv, Sv, VP)[:, :, :V]
    loss2d = _make_loss(NTOK)(part.reshape(NW, L))
    return logits, loss2d[0, 0]


# speedup vs baseline: 1.2998x; 1.1489x over previous
import functools

import jax
import jax.numpy as jnp
from jax import lax
from jax.experimental import pallas as pl
from jax.experimental.pallas import tpu as pltpu
from jax.experimental.pallas import tpu_sc as plsc

NC = 2
NS = 16
L = 16
NW = NC * NS

LANE = 128
VP = 1024
SUB = VP // LANE


def _mtab_body(emb_ref, w_ref, b_ref, m8_ref, lse_ref):
    m = jax.lax.dot_general(
        emb_ref[...], w_ref[...],
        (((1,), (1,)), ((), ())),
        preferred_element_type=jnp.float32,
    ) + b_ref[...]
    V = m.shape[1]
    for j in range(SUB):
        w = min(LANE, V - j * LANE)
        m8_ref[:, j, :w] = m[:, j * LANE:j * LANE + w]
    mx = jnp.max(m, axis=1, keepdims=True)
    lse_ref[...] = mx + jnp.log(jnp.sum(jnp.exp(m - mx), axis=1, keepdims=True))


def _make_mtab(V):
    return pl.pallas_call(
        _mtab_body,
        out_shape=(
            jax.ShapeDtypeStruct((V, SUB, LANE), jnp.float32),
            jax.ShapeDtypeStruct((V, 1), jnp.float32),
        ),
    )


def _make_gather(V, NTOK):
    TPT = NTOK // NW
    CH = 64
    assert TPT % CH == 0 and CH % L == 0
    mesh = plsc.VectorSubcoreMesh(core_axis_name="c", subcore_axis_name="s")

    @functools.partial(
        pl.kernel,
        mesh=mesh,
        compiler_params=pltpu.CompilerParams(
            use_tc_tiling_on_sc=False, needs_layout_passes=False),
        out_type=(
            jax.ShapeDtypeStruct((NTOK, SUB, LANE), jnp.float32),
            jax.ShapeDtypeStruct((NW * L,), jnp.float32),
        ),
        scratch_types=[
            pltpu.VMEM((TPT,), jnp.int32),
            pltpu.VMEM((TPT,), jnp.int32),
            pltpu.VMEM((V,), jnp.float32),
            pltpu.VMEM((CH, SUB, LANE), jnp.float32),
            pltpu.VMEM((L,), jnp.float32),
            pltpu.SemaphoreType.DMA,
        ],
    )
    def gather_k(m8_hbm, idx_hbm, tgt_hbm, lse_hbm, out_hbm, part_hbm,
                 idx_v, tgt_v, lse_v, rows_v, acc_v, sem):
        wid = lax.axis_index("s") * NC + lax.axis_index("c")
        base = wid * TPT
        pltpu.sync_copy(idx_hbm.at[pl.ds(base, TPT)], idx_v)
        pltpu.sync_copy(tgt_hbm.at[pl.ds(base, TPT)], tgt_v)
        pltpu.sync_copy(lse_hbm, lse_v)
        acc = jnp.zeros((L,), jnp.float32)
        for c in range(TPT // CH):
            pltpu.async_copy(
                m8_hbm.at[idx_v.at[pl.ds(c * CH, CH)]], rows_v, sem).wait()
            pltpu.sync_copy(rows_v, out_hbm.at[pl.ds(base + c * CH, CH)])
            for g in range(CH // L):
                off = c * CH + g * L
                toks = idx_v[pl.ds(off, L)]
                tgts = tgt_v[pl.ds(off, L)]
                rid = lax.iota(jnp.int32, L) + (g * L)
                tlogit = plsc.load_gather(
                    rows_v, [rid, tgts >> 7, tgts & (LANE - 1)])
                ltok = plsc.load_gather(lse_v, [toks])
                acc = acc + (ltok - tlogit)
        acc_v[...] = acc
        pltpu.sync_copy(acc_v, part_hbm.at[pl.ds(wid * L, L)])

    return gather_k


def _fin_body(in_ref, out_ref, *, V):
    rows = lax.broadcasted_iota(jnp.int32, (LANE, LANE), 0)
    cols = lax.broadcasted_iota(jnp.int32, (LANE, LANE), 1)
    ident = jnp.where(rows == cols, 1.0, 0.0).astype(jnp.float32)
    for j in range(SUB):
        w = min(LANE, V - j * LANE)
        xt = jax.lax.dot_general(
            ident, in_ref[:, j, :], (((1,), (1,)), ((), ())),
            preferred_element_type=jnp.float32)
        out_ref[0, pl.ds(j * LANE, w), :] = xt[:w, :]


def _make_finisher(V, Bv, Sv):
    TB = 256
    grid = (Bv, Sv // TB)
    return pl.pallas_call(
        functools.partial(_fin_body, V=V),
        grid=grid,
        in_specs=[pl.BlockSpec(
            (TB, SUB, LANE), lambda b, t: (b * (Sv // TB) + t, 0, 0))],
        out_specs=pl.BlockSpec((1, V, TB), lambda b, t: (b, 0, t)),
        out_shape=jax.ShapeDtypeStruct((Bv, V, Sv), jnp.float32),
    )


def _loss_body(p_ref, o_ref, *, ntok):
    o_ref[...] = jnp.sum(p_ref[...], keepdims=True) * (1.0 / ntok)


def _make_loss(ntok):
    return pl.pallas_call(
        functools.partial(_loss_body, ntok=ntok),
        out_shape=jax.ShapeDtypeStruct((1, 1), jnp.float32),
    )


def kernel(input, target, emb_table, W, b):
    Bv, Sv = input.shape
    V, E = emb_table.shape
    NTOK = Bv * Sv

    m8, lse = _make_mtab(V)(emb_table, W, b.reshape(1, V))
    idx = input.reshape(NTOK)
    tgt = target.reshape(NTOK)
    out8, part = _make_gather(V, NTOK)(m8, idx, tgt, lse.reshape(V))
    logits_t = _make_finisher(V, Bv, Sv)(out8)
    logits = jnp.swapaxes(logits_t, 1, 2)
    loss2d = _make_loss(NTOK)(part.reshape(NW, L))
    return logits, loss2d[0, 0]


# speedup vs baseline: 1.4574x; 1.1212x over previous
import functools

import jax
import jax.numpy as jnp
from jax import lax
from jax.experimental import pallas as pl
from jax.experimental.pallas import tpu as pltpu
from jax.experimental.pallas import tpu_sc as plsc

NC = 2
NS = 16
L = 16
NW = NC * NS

LANE = 128
VP = 1024
SUB = VP // LANE


def _mtab_body(emb_ref, w_ref, b_ref, m8_ref, lse_ref):
    m = jax.lax.dot_general(
        emb_ref[...], w_ref[...],
        (((1,), (1,)), ((), ())),
        preferred_element_type=jnp.float32,
    ) + b_ref[...]
    V = m.shape[1]
    for j in range(SUB):
        w = min(LANE, V - j * LANE)
        m8_ref[:, j, :w] = m[:, j * LANE:j * LANE + w]
    mx = jnp.max(m, axis=1, keepdims=True)
    lse_ref[...] = mx + jnp.log(jnp.sum(jnp.exp(m - mx), axis=1, keepdims=True))


def _make_mtab(V):
    return pl.pallas_call(
        _mtab_body,
        out_shape=(
            jax.ShapeDtypeStruct((V, SUB, LANE), jnp.float32),
            jax.ShapeDtypeStruct((V, 1), jnp.float32),
        ),
    )


def _make_gather(V, NTOK):
    TPT = NTOK // NW
    CH = 64
    assert TPT % CH == 0 and CH % L == 0
    mesh = plsc.VectorSubcoreMesh(core_axis_name="c", subcore_axis_name="s")

    @functools.partial(
        pl.kernel,
        mesh=mesh,
        compiler_params=pltpu.CompilerParams(
            use_tc_tiling_on_sc=False, needs_layout_passes=False),
        out_type=(
            jax.ShapeDtypeStruct((NTOK, SUB, LANE), jnp.float32),
            jax.ShapeDtypeStruct((NW * L,), jnp.float32),
        ),
        scratch_types=[
            pltpu.VMEM((TPT,), jnp.int32),
            pltpu.VMEM((TPT,), jnp.int32),
            pltpu.VMEM((V,), jnp.float32),
            pltpu.VMEM((CH, SUB, LANE), jnp.float32),
            pltpu.VMEM((L,), jnp.float32),
            pltpu.SemaphoreType.DMA,
        ],
    )
    def gather_k(m8_hbm, idx_hbm, tgt_hbm, lse_hbm, out_hbm, part_hbm,
                 idx_v, tgt_v, lse_v, rows_v, acc_v, sem):
        wid = lax.axis_index("s") * NC + lax.axis_index("c")
        base = wid * TPT
        pltpu.sync_copy(idx_hbm.at[pl.ds(base, TPT)], idx_v)
        pltpu.sync_copy(tgt_hbm.at[pl.ds(base, TPT)], tgt_v)
        pltpu.sync_copy(lse_hbm, lse_v)
        acc = jnp.zeros((L,), jnp.float32)
        for c in range(TPT // CH):
            pltpu.async_copy(
                m8_hbm.at[idx_v.at[pl.ds(c * CH, CH)]], rows_v, sem).wait()
            pltpu.sync_copy(rows_v, out_hbm.at[pl.ds(base + c * CH, CH)])
            for g in range(CH // L):
                off = c * CH + g * L
                toks = idx_v[pl.ds(off, L)]
                tgts = tgt_v[pl.ds(off, L)]
                rid = lax.iota(jnp.int32, L) + (g * L)
                tlogit = plsc.load_gather(
                    rows_v, [rid, tgts >> 7, tgts & (LANE - 1)])
                ltok = plsc.load_gather(lse_v, [toks])
                acc = acc + (ltok - tlogit)
        acc_v[...] = acc
        pltpu.sync_copy(acc_v, part_hbm.at[pl.ds(wid * L, L)])

    return gather_k


def _fin_body(in_ref, out_ref, *, V):
    for j in range(SUB):
        w = min(LANE, V - j * LANE)
        xt = jax.lax.transpose(in_ref[:, j, :], (1, 0))
        out_ref[0, pl.ds(j * LANE, w), :] = xt[:w, :]


def _make_finisher(V, Bv, Sv):
    TB = 512
    grid = (Bv, Sv // TB)
    return pl.pallas_call(
        functools.partial(_fin_body, V=V),
        grid=grid,
        in_specs=[pl.BlockSpec(
            (TB, SUB, LANE), lambda b, t: (b * (Sv // TB) + t, 0, 0))],
        out_specs=pl.BlockSpec((1, V, TB), lambda b, t: (b, 0, t)),
        out_shape=jax.ShapeDtypeStruct((Bv, V, Sv), jnp.float32),
    )


def _loss_body(p_ref, o_ref, *, ntok):
    o_ref[...] = jnp.sum(p_ref[...], keepdims=True) * (1.0 / ntok)


def _make_loss(ntok):
    return pl.pallas_call(
        functools.partial(_loss_body, ntok=ntok),
        out_shape=jax.ShapeDtypeStruct((1, 1), jnp.float32),
    )


def kernel(input, target, emb_table, W, b):
    Bv, Sv = input.shape
    V, E = emb_table.shape
    NTOK = Bv * Sv

    m8, lse = _make_mtab(V)(emb_table, W, b.reshape(1, V))
    idx = input.reshape(NTOK)
    tgt = target.reshape(NTOK)
    out8, part = _make_gather(V, NTOK)(m8, idx, tgt, lse.reshape(V))
    logits_t = _make_finisher(V, Bv, Sv)(out8)
    logits = jnp.swapaxes(logits_t, 1, 2)
    loss2d = _make_loss(NTOK)(part.reshape(NW, L))
    return logits, loss2d[0, 0]


# speedup vs baseline: 1.4756x; 1.0125x over previous
import functools

import jax
import jax.numpy as jnp
from jax import lax
from jax.experimental import pallas as pl
from jax.experimental.pallas import tpu as pltpu
from jax.experimental.pallas import tpu_sc as plsc

NC = 2
NS = 16
L = 16
NW = NC * NS

LANE = 128
VP = 1024
SUB = VP // LANE


def _mtab_body(emb_ref, w_ref, b_ref, m8_ref, lse_ref):
    m = jax.lax.dot_general(
        emb_ref[...], w_ref[...],
        (((1,), (1,)), ((), ())),
        preferred_element_type=jnp.float32,
    ) + b_ref[...]
    V = m.shape[1]
    for j in range(SUB):
        w = min(LANE, V - j * LANE)
        m8_ref[:, j, :w] = m[:, j * LANE:j * LANE + w]
    mx = jnp.max(m, axis=1, keepdims=True)
    lse_ref[...] = mx + jnp.log(jnp.sum(jnp.exp(m - mx), axis=1, keepdims=True))


def _make_mtab(V):
    return pl.pallas_call(
        _mtab_body,
        out_shape=(
            jax.ShapeDtypeStruct((V, SUB, LANE), jnp.float32),
            jax.ShapeDtypeStruct((V, 1), jnp.float32),
        ),
    )


def _make_gather(V, NTOK):
    TPT = NTOK // NW
    CH = 32
    assert TPT % CH == 0 and CH % L == 0
    mesh = plsc.VectorSubcoreMesh(core_axis_name="c", subcore_axis_name="s")

    @functools.partial(
        pl.kernel,
        mesh=mesh,
        compiler_params=pltpu.CompilerParams(
            use_tc_tiling_on_sc=False, needs_layout_passes=False),
        out_type=(
            jax.ShapeDtypeStruct((NTOK, SUB, LANE), jnp.float32),
            jax.ShapeDtypeStruct((NW * L,), jnp.float32),
        ),
        scratch_types=[
            pltpu.VMEM((TPT,), jnp.int32),
            pltpu.VMEM((TPT,), jnp.int32),
            pltpu.VMEM((V,), jnp.float32),
            pltpu.VMEM((CH, SUB, LANE), jnp.float32),
            pltpu.VMEM((CH, SUB, LANE), jnp.float32),
            pltpu.VMEM((L,), jnp.float32),
            pltpu.SemaphoreType.DMA,
            pltpu.SemaphoreType.DMA,
            pltpu.SemaphoreType.DMA,
            pltpu.SemaphoreType.DMA,
        ],
    )
    def gather_k(m8_hbm, idx_hbm, tgt_hbm, lse_hbm, out_hbm, part_hbm,
                 idx_v, tgt_v, lse_v, rows0_v, rows1_v, acc_v,
                 gsem0, gsem1, osem0, osem1):
        wid = lax.axis_index("s") * NC + lax.axis_index("c")
        base = wid * TPT
        pltpu.sync_copy(idx_hbm.at[pl.ds(base, TPT)], idx_v)
        pltpu.sync_copy(tgt_hbm.at[pl.ds(base, TPT)], tgt_v)
        pltpu.sync_copy(lse_hbm, lse_v)
        rows = (rows0_v, rows1_v)
        gsem = (gsem0, gsem1)
        osem = (osem0, osem1)
        NCH = TPT // CH
        acc = jnp.zeros((L,), jnp.float32)
        gcp = [None] * NCH
        ocp = [None] * NCH
        gcp[0] = pltpu.async_copy(
            m8_hbm.at[idx_v.at[pl.ds(0, CH)]], rows[0], gsem[0])
        for c in range(NCH):
            k = c % 2
            if c + 1 < NCH:
                if c >= 1:
                    ocp[c - 1].wait()
                gcp[c + 1] = pltpu.async_copy(
                    m8_hbm.at[idx_v.at[pl.ds((c + 1) * CH, CH)]],
                    rows[1 - k], gsem[1 - k])
            gcp[c].wait()
            ocp[c] = pltpu.async_copy(
                rows[k], out_hbm.at[pl.ds(base + c * CH, CH)], osem[k])
            for g in range(CH // L):
                off = c * CH + g * L
                toks = idx_v[pl.ds(off, L)]
                tgts = tgt_v[pl.ds(off, L)]
                rid = lax.iota(jnp.int32, L) + (g * L)
                tlogit = plsc.load_gather(
                    rows[k], [rid, tgts >> 7, tgts & (LANE - 1)])
                ltok = plsc.load_gather(lse_v, [toks])
                acc = acc + (ltok - tlogit)
        ocp[NCH - 2].wait()
        ocp[NCH - 1].wait()
        acc_v[...] = acc
        pltpu.sync_copy(acc_v, part_hbm.at[pl.ds(wid * L, L)])

    return gather_k


def _fin_body(in_ref, out_ref, *, V):
    for j in range(SUB):
        w = min(LANE, V - j * LANE)
        xt = jax.lax.transpose(in_ref[:, j, :], (1, 0))
        out_ref[0, pl.ds(j * LANE, w), :] = xt[:w, :]


def _make_finisher(V, Bv, Sv):
    TB = 512
    grid = (Bv, Sv // TB)
    return pl.pallas_call(
        functools.partial(_fin_body, V=V),
        grid=grid,
        in_specs=[pl.BlockSpec(
            (TB, SUB, LANE), lambda b, t: (b * (Sv // TB) + t, 0, 0))],
        out_specs=pl.BlockSpec((1, V, TB), lambda b, t: (b, 0, t)),
        out_shape=jax.ShapeDtypeStruct((Bv, V, Sv), jnp.float32),
    )


def _loss_body(p_ref, o_ref, *, ntok):
    o_ref[...] = jnp.sum(p_ref[...], keepdims=True) * (1.0 / ntok)


def _make_loss(ntok):
    return pl.pallas_call(
        functools.partial(_loss_body, ntok=ntok),
        out_shape=jax.ShapeDtypeStruct((1, 1), jnp.float32),
    )


def kernel(input, target, emb_table, W, b):
    Bv, Sv = input.shape
    V, E = emb_table.shape
    NTOK = Bv * Sv

    m8, lse = _make_mtab(V)(emb_table, W, b.reshape(1, V))
    idx = input.reshape(NTOK)
    tgt = target.reshape(NTOK)
    out8, part = _make_gather(V, NTOK)(m8, idx, tgt, lse.reshape(V))
    logits_t = _make_finisher(V, Bv, Sv)(out8)
    logits = jnp.swapaxes(logits_t, 1, 2)
    loss2d = _make_loss(NTOK)(part.reshape(NW, L))
    return logits, loss2d[0, 0]
